# baseline (speedup 1.0000x reference)
import functools

import jax
import jax.numpy as jnp
from jax import lax
from jax.experimental import pallas as pl
from jax.experimental.pallas import tpu as pltpu
from jax.experimental.pallas import tpu_sc as plsc


def _precompute_body(scalars_ref, wa_ref, wb_ref, b1_ref, pa_ref, pb_ref):
    s = scalars_ref[...]
    pa_ref[...] = jnp.dot(s, wa_ref[...], preferred_element_type=jnp.float32)
    pb_ref[...] = (
        jnp.dot(s, wb_ref[...], preferred_element_type=jnp.float32)
        + b1_ref[...]
    )


def _precompute(scalars, wa, wb, b1):
    n, _ = scalars.shape
    h = wa.shape[1]
    return pl.pallas_call(
        _precompute_body,
        out_shape=[
            jax.ShapeDtypeStruct((n, h), jnp.float32),
            jax.ShapeDtypeStruct((n, h), jnp.float32),
        ],
    )(scalars, wa, wb, b1.reshape(1, h))


def _sc_gather(pa, pb, snd, rcv, elen, w1c, crows=200, mblk=1600):
    n, h = pa.shape
    e = snd.shape[0]
    e2 = e // 2
    info = plsc.get_sparse_core_info()
    nc, ns = info.num_cores, info.num_subcores
    nw = nc * ns
    assert e2 % nw == 0
    rpw = e2 // nw
    assert rpw % crows == 0 and mblk % crows == 0
    steps = rpw // crows
    assert steps % 2 == 1
    nv = h // 16
    gfull = crows // 16
    gtail = crows - 16 * gfull
    parts = ((0, gfull // 2, False), (gfull // 2, gfull - gfull // 2, True))
    mesh = plsc.VectorSubcoreMesh(core_axis_name="c", subcore_axis_name="s")

    croff = ((crows + 15) // 16) * 16
    buf_set = [
        pltpu.VMEM((2 * crows,), jnp.int32),
        pltpu.VMEM((2 * crows,), jnp.int32),
        pltpu.VMEM((croff + crows,), jnp.float32),
        pltpu.VMEM((2 * crows, h), jnp.float32),
        pltpu.VMEM((2 * crows, h), jnp.float32),
        pltpu.SemaphoreType.DMA,
        pltpu.SemaphoreType.DMA,
    ]
    out_rows = 16 * (gfull - gfull // 2) + gtail

    @functools.partial(
        pl.kernel,
        mesh=mesh,
        compiler_params=pltpu.CompilerParams(use_tc_tiling_on_sc=False),
        out_type=jax.ShapeDtypeStruct((e2, 2 * h), jnp.float32),
        scratch_types=[
            pltpu.VMEM((h,), jnp.float32),
            pltpu.VMEM((out_rows, 2 * h), jnp.float32),
        ] + buf_set + buf_set,
    )
    def k(pa_hbm, pb_hbm, snd_hbm, rcv_hbm, len_hbm, w1c_hbm, x2_hbm,
          w1c_v, out_v, *bufs):
        sets = (bufs[:7], bufs[7:])
        wid = lax.axis_index("s") * nc + lax.axis_index("c")
        base0 = wid * rpw
        pltpu.sync_copy(w1c_hbm, w1c_v)
        wvecs = tuple(w1c_v[pl.ds(16 * kk, 16)] for kk in range(nv))
        zl = lax.broadcasted_iota(jnp.int32, (16,), 0) * 0

        def edge_bases(c):
            r = base0 + c * crows
            blk = r // mblk
            r0 = r - blk * mblk
            el = 2 * mblk * blk + r0
            return r, el, el + mblk

        def issue(c, s):
            ia, ib, ln, ba, bb, sema, semb = s
            _, el, er = edge_bases(c)
            cps = [
                pltpu.async_copy(snd_hbm.at[pl.ds(el, crows)],
                                 ia.at[pl.ds(0, crows)], sema),
                pltpu.async_copy(snd_hbm.at[pl.ds(er, crows)],
                                 ia.at[pl.ds(crows, crows)], sema),
                pltpu.async_copy(rcv_hbm.at[pl.ds(el, crows)],
                                 ib.at[pl.ds(0, crows)], sema),
                pltpu.async_copy(rcv_hbm.at[pl.ds(er, crows)],
                                 ib.at[pl.ds(crows, crows)], sema),
                pltpu.async_copy(len_hbm.at[pl.ds(el, crows)],
                                 ln.at[pl.ds(0, crows)], sema),
                pltpu.async_copy(len_hbm.at[pl.ds(er, crows)],
                                 ln.at[pl.ds(croff, crows)], sema),
            ]
            for cp in cps:
                cp.wait()
            pltpu.async_copy(pa_hbm.at[ia], ba, sema)
            pltpu.async_copy(pb_hbm.at[ib], bb, semb)

        def row_combine(jbuf, jout, sl_, sr_, s, cw):
            ia, ib, ln, ba, bb, sema, semb = s
            for q in range(nv):
                sl = pl.ds(16 * q, 16)
                out_v[jout, sl] = (
                    ba[jbuf, sl] + bb[jbuf, sl] + sl_ * cw[q])
                sl2 = pl.ds(h + 16 * q, 16)
                out_v[jout, sl2] = (
                    ba[crows + jbuf, sl] + bb[crows + jbuf, sl]
                    + sr_ * cw[q])

        def finish(c, s, ws):
            ia, ib, ln, ba, bb, sema, semb = s
            r, _, _ = edge_bases(c)
            pltpu.make_async_copy(pa_hbm.at[ia], ba, sema).wait()
            pltpu.make_async_copy(pb_hbm.at[ib], bb, semb).wait()

            for g0, ng, has_tail in parts:
                def combine(g, cw):
                    rb = 16 * (g0 + g)
                    lvl = ln[pl.ds(rb, 16)]
                    lvr = ln[pl.ds(croff + rb, 16)]
                    for t in range(16):
                        row_combine(rb + t, 16 * g + t,
                                    jnp.take(lvl, zl + t),
                                    jnp.take(lvr, zl + t), s, cw)
                    return cw

                lax.fori_loop(0, ng, combine, ws)
                nrows = 16 * ng
                if has_tail and gtail:
                    lvl = ln[pl.ds(crows - 16, 16)]
                    lvr = ln[pl.ds(croff + crows - 16, 16)]
                    for t in range(gtail):
                        row_combine(16 * gfull + t, 16 * ng + t,
                                    jnp.take(lvl, zl + (16 - gtail + t)),
                                    jnp.take(lvr, zl + (16 - gtail + t)),
                                    s, ws)
                    nrows += gtail
                pltpu.sync_copy(
                    out_v.at[pl.ds(0, nrows)],
                    x2_hbm.at[pl.ds(r + 16 * g0, nrows)])

        issue(0, sets[0])

        def body(k2, ws):
            c0 = 2 * k2
            issue(c0 + 1, sets[1])
            finish(c0, sets[0], ws)

            @pl.when(c0 + 2 < steps)
            def _():
                issue(c0 + 2, sets[0])

            finish(c0 + 1, sets[1], ws)
            return ws

        wvecs = lax.fori_loop(0, steps // 2, body, wvecs)
        finish(steps - 1, sets[0], wvecs)

    return k(pa, pb, snd, rcv, elen, w1c)


def _mlp_body(x_ref, w2d_ref, b2d_ref, out_ref):
    br, w = x_ref.shape
    h = w // 2
    x = x_ref[...]
    hh = x * jax.nn.sigmoid(x)
    o2 = (
        jnp.dot(hh, w2d_ref[...], preferred_element_type=jnp.float32)
        + b2d_ref[...]
    )
    out_ref[0:br, :] = o2[:, :h]
    out_ref[br:2 * br, :] = o2[:, h:]


def _edge_mlp(x2, w2d, b2d, block=1600):
    e2, w = x2.shape
    h = w // 2
    assert e2 % block == 0
    grid = (e2 // block,)
    row = lambda i: (i, 0)
    full = lambda i: (0, 0)
    return pl.pallas_call(
        _mlp_body,
        grid=grid,
        in_specs=[
            pl.BlockSpec((block, w), row),
            pl.BlockSpec((w, w), full),
            pl.BlockSpec((1, w), full),
        ],
        out_specs=pl.BlockSpec((2 * block, h), row),
        out_shape=jax.ShapeDtypeStruct((2 * e2, h), jnp.float32),
    )(x2, w2d, b2d)


def kernel(scalars, edge_index, edge_len, W1, b1, W2, b2):
    n, d = scalars.shape
    h = W1.shape[1]
    wa = W1[:d]
    wb = W1[d:2 * d]
    pa, pb = _precompute(scalars, wa, wb, b1)
    snd = edge_index[0]
    rcv = edge_index[1]
    x2 = _sc_gather(pa, pb, snd, rcv, edge_len, W1[2 * d])
    zero = jnp.zeros((h, h), jnp.float32)
    w2d = jnp.block([[W2, zero], [zero, W2]])
    b2d = jnp.concatenate([b2.reshape(1, h), b2.reshape(1, h)], axis=1)
    return _edge_mlp(x2, w2d, b2d)


# speedup vs baseline: 1.8379x; 1.8379x over previous
import functools

import jax
import jax.numpy as jnp
from jax import lax
from jax.experimental import pallas as pl
from jax.experimental.pallas import tpu as pltpu
from jax.experimental.pallas import tpu_sc as plsc


def _precompute_body(scalars_ref, wa_ref, wb_ref, b1_ref, pa_ref, pb_ref):
    s = scalars_ref[...]
    pa_ref[...] = jnp.dot(s, wa_ref[...], preferred_element_type=jnp.float32)
    pb_ref[...] = (
        jnp.dot(s, wb_ref[...], preferred_element_type=jnp.float32)
        + b1_ref[...]
    )


def _precompute(scalars, wa, wb, b1):
    n, _ = scalars.shape
    h = wa.shape[1]
    return pl.pallas_call(
        _precompute_body,
        out_shape=[
            jax.ShapeDtypeStruct((n, h), jnp.float32),
            jax.ShapeDtypeStruct((n, h), jnp.float32),
        ],
    )(scalars, wa, wb, b1.reshape(1, h))


def _sc_gather(pa, pb, snd, rcv, elen, w1c, chunk=400):
    n, h = pa.shape
    e = snd.shape[0]
    info = plsc.get_sparse_core_info()
    nc, ns = info.num_cores, info.num_subcores
    nw = nc * ns
    assert e % nw == 0
    epw = e // nw
    assert epw % chunk == 0 and chunk % 16 == 0
    steps = epw // chunk
    assert steps % 2 == 1
    nv = h // 16
    mesh = plsc.VectorSubcoreMesh(core_axis_name="c", subcore_axis_name="s")

    buf_set = [
        pltpu.VMEM((chunk,), jnp.int32),
        pltpu.VMEM((chunk,), jnp.int32),
        pltpu.VMEM((chunk,), jnp.float32),
        pltpu.VMEM((chunk, h), jnp.float32),
        pltpu.VMEM((chunk, h), jnp.float32),
        pltpu.SemaphoreType.DMA,
        pltpu.SemaphoreType.DMA,
    ]

    @functools.partial(
        pl.kernel,
        mesh=mesh,
        compiler_params=pltpu.CompilerParams(use_tc_tiling_on_sc=False),
        out_type=jax.ShapeDtypeStruct((e // 2, 2 * h), jnp.float32),
        scratch_types=[
            pltpu.VMEM((h,), jnp.float32),
            pltpu.VMEM(((chunk // 32 + 1) * 8, 2 * h), jnp.float32),
        ] + buf_set + buf_set,
    )
    def k(pa_hbm, pb_hbm, snd_hbm, rcv_hbm, len_hbm, w1c_hbm, x2_hbm,
          w1c_v, out_v, *bufs):
        sets = (bufs[:7], bufs[7:])
        wid = lax.axis_index("s") * nc + lax.axis_index("c")
        base0 = wid * epw
        pltpu.sync_copy(w1c_hbm, w1c_v)
        wvecs = tuple(w1c_v[pl.ds(16 * kk, 16)] for kk in range(nv))

        def issue(c, s):
            idxa_v, idxb_v, len_v, bufa_v, bufb_v, sema, semb = s
            base = base0 + c * chunk
            cps = [
                pltpu.async_copy(snd_hbm.at[pl.ds(base, chunk)], idxa_v,
                                 sema),
                pltpu.async_copy(rcv_hbm.at[pl.ds(base, chunk)], idxb_v,
                                 sema),
                pltpu.async_copy(len_hbm.at[pl.ds(base, chunk)], len_v,
                                 sema),
            ]
            for cp in cps:
                cp.wait()
            pltpu.async_copy(pa_hbm.at[idxa_v], bufa_v, sema)
            pltpu.async_copy(pb_hbm.at[idxb_v], bufb_v, semb)

        def finish(c, s, ws):
            idxa_v, idxb_v, len_v, bufa_v, bufb_v, sema, semb = s
            base = base0 + c * chunk
            pltpu.make_async_copy(pa_hbm.at[idxa_v], bufa_v, sema).wait()
            pltpu.make_async_copy(pb_hbm.at[idxb_v], bufb_v, semb).wait()

            ngroups = chunk // 16
            for g0, ng in ((0, ngroups // 2), (ngroups // 2,
                                               ngroups - ngroups // 2)):
                def combine(g, cw):
                    ebase = 16 * (g0 + g)
                    lv = len_v[pl.ds(ebase, 16)]
                    for t in range(8):
                        e0 = ebase + 2 * t
                        e1 = ebase + 2 * t + 1
                        j = 8 * g + t
                        s0 = lv[2 * t]
                        s1 = lv[2 * t + 1]
                        for q in range(nv):
                            sl = pl.ds(16 * q, 16)
                            out_v[j, sl] = (
                                bufa_v[e0, sl] + bufb_v[e0, sl] + s0 * cw[q])
                            sl2 = pl.ds(h + 16 * q, 16)
                            out_v[j, sl2] = (
                                bufa_v[e1, sl] + bufb_v[e1, sl] + s1 * cw[q])
                    return cw

                lax.fori_loop(0, ng, combine, ws)
                pltpu.sync_copy(
                    out_v.at[pl.ds(0, ng * 8)],
                    x2_hbm.at[pl.ds(base // 2 + g0 * 8, ng * 8)])

        issue(0, sets[0])

        def body(k2, ws):
            c0 = 2 * k2
            issue(c0 + 1, sets[1])
            finish(c0, sets[0], ws)

            @pl.when(c0 + 2 < steps)
            def _():
                issue(c0 + 2, sets[0])

            finish(c0 + 1, sets[1], ws)
            return ws

        wvecs = lax.fori_loop(0, steps // 2, body, wvecs)
        finish(steps - 1, sets[0], wvecs)

    return k(pa, pb, snd, rcv, elen, w1c)


def _sc_emit(x2p, rows=500):
    e2, w = x2p.shape
    info = plsc.get_sparse_core_info()
    nw = info.num_cores * info.num_subcores
    nc = info.num_cores
    assert e2 % (nw * rows) == 0
    rpw = e2 // nw
    steps = rpw // rows
    mesh = plsc.VectorSubcoreMesh(core_axis_name="c", subcore_axis_name="s")

    @functools.partial(
        pl.kernel,
        mesh=mesh,
        compiler_params=pltpu.CompilerParams(use_tc_tiling_on_sc=False),
        out_type=jax.ShapeDtypeStruct((e2, w), jnp.float32),
        scratch_types=[
            pltpu.VMEM((rows, w), jnp.float32),
            pltpu.SemaphoreType.DMA,
        ],
    )
    def k(x_hbm, out_hbm, buf_v, sem):
        wid = lax.axis_index("s") * nc + lax.axis_index("c")
        base0 = wid * rpw

        def body(kk, carry):
            r = base0 + kk * rows
            pltpu.sync_copy(x_hbm.at[pl.ds(r, rows)], buf_v)
            pltpu.sync_copy(buf_v, out_hbm.at[pl.ds(r, rows)])
            return carry

        lax.fori_loop(0, steps, body, 0)

    return k(x2p)


def _mlp_body(x_ref, w2d_ref, b2d_ref, out_ref):
    x = x_ref[...]
    hh = x * jax.nn.sigmoid(x)
    out_ref[...] = (
        jnp.dot(hh, w2d_ref[...], preferred_element_type=jnp.float32)
        + b2d_ref[...]
    )


def _edge_mlp(x2, w2d, b2d, block=1600):
    e2, w = x2.shape
    assert e2 % block == 0
    grid = (e2 // block,)
    row = lambda i: (i, 0)
    full = lambda i: (0, 0)
    return pl.pallas_call(
        _mlp_body,
        grid=grid,
        in_specs=[
            pl.BlockSpec((block, w), row),
            pl.BlockSpec((w, w), full),
            pl.BlockSpec((1, w), full),
        ],
        out_specs=pl.BlockSpec((block, w), row),
        out_shape=jax.ShapeDtypeStruct((e2, w), jnp.float32),
    )(x2, w2d, b2d)


def kernel(scalars, edge_index, edge_len, W1, b1, W2, b2):
    n, d = scalars.shape
    h = W1.shape[1]
    e = edge_index.shape[1]
    wa = W1[:d]
    wb = W1[d:2 * d]
    pa, pb = _precompute(scalars, wa, wb, b1)
    snd = edge_index[0]
    rcv = edge_index[1]
    x2 = _sc_gather(pa, pb, snd, rcv, edge_len, W1[2 * d])
    zero = jnp.zeros((h, h), jnp.float32)
    w2d = jnp.block([[W2, zero], [zero, W2]])
    b2d = jnp.concatenate([b2.reshape(1, h), b2.reshape(1, h)], axis=1)
    return _edge_mlp(x2, w2d, b2d)
